# Initial kernel scaffold; baseline (speedup 1.0000x reference)
import jax
import jax.numpy as jnp
from jax.experimental import pallas as pl


def kernel(ngrams, ngrams_weights, emb, W1, b1, W2, b2):
    raise NotImplementedError("write your pallas kernel here")


# speedup vs baseline: 3.2484x; 3.2484x over previous
import functools

import jax
import jax.numpy as jnp
from jax import lax
from jax.experimental import pallas as pl
from jax.experimental.pallas import tpu as pltpu
from jax.experimental.pallas import tpu_sc as plsc

_VOCAB = 1000000
_EMBED = 32
_LABELS = 107
_ORDER = 3
_HASHES = 20
_BATCH = 16384
_PER_ROW = _ORDER * _HASHES

_NC = 2
_NS = 16
_NW = _NC * _NS
_BPW = _BATCH // _NW
_CB = 16
_NCHUNK = _BPW // _CB
_IPC = _CB * _PER_ROW
_GS = 120
_NG = _IPC // _GS


def _sc_body(idx_hbm, w_hbm, emb_hbm, out_hbm, idx_v, w_v, rows_v, out_v, sem):
    wid = lax.axis_index("s") * _NC + lax.axis_index("c")

    def chunk_body(g, carry):
        row0 = wid * _BPW + g * _CB
        flat0 = row0 * _PER_ROW

        pltpu.sync_copy(idx_hbm.at[pl.ds(flat0, _IPC)], idx_v)
        pltpu.sync_copy(w_hbm.at[pl.ds(flat0, _IPC)], w_v)

        copies = [
            pltpu.async_copy(
                emb_hbm.at[idx_v.at[pl.ds(j * _GS, _GS)]],
                rows_v.at[pl.ds(j * _GS, _GS)],
                sem,
            )
            for j in range(_NG)
        ]
        for c in copies:
            c.wait()

        def b_body(b, carry2):
            for o in range(_ORDER):
                acc0 = jnp.zeros((16,), jnp.float32)
                acc1 = jnp.zeros((16,), jnp.float32)
                pos0 = b * _PER_ROW + o * _HASHES
                wv0 = w_v[pl.ds(pos0, 16)]
                wv1 = w_v[pl.ds(pos0 + 4, 16)]
                for h in range(_HASHES):
                    pos = pos0 + h
                    w = wv0[h] if h < 16 else wv1[h - 4]
                    wb = jnp.full((16,), w, jnp.float32)
                    r0 = rows_v[pos, pl.ds(0, 16)]
                    r1 = rows_v[pos, pl.ds(16, 16)]
                    acc0 = acc0 + wb * r0
                    acc1 = acc1 + wb * r1
                out_v[b, pl.ds(o * _EMBED, 16)] = acc0
                out_v[b, pl.ds(o * _EMBED + 16, 16)] = acc1
            return carry2

        lax.fori_loop(0, _CB, b_body, 0)
        pltpu.sync_copy(out_v, out_hbm.at[pl.ds(row0, _CB)])
        return carry

    lax.fori_loop(0, _NCHUNK, chunk_body, 0)


@functools.partial(jax.jit, static_argnames=())
def _sc_gather(idx_flat, w_flat, emb):
    mesh = plsc.VectorSubcoreMesh(core_axis_name="c", subcore_axis_name="s")
    k = functools.partial(
        pl.kernel,
        mesh=mesh,
        compiler_params=pltpu.CompilerParams(use_tc_tiling_on_sc=False),
        out_type=jax.ShapeDtypeStruct((_BATCH, _ORDER * _EMBED), jnp.float32),
        scratch_types=[
            pltpu.VMEM((_IPC,), jnp.int32),
            pltpu.VMEM((_IPC,), jnp.float32),
            pltpu.VMEM((_IPC, _EMBED), jnp.float32),
            pltpu.VMEM((_CB, _ORDER * _EMBED), jnp.float32),
            pltpu.SemaphoreType.DMA,
        ],
    )(_sc_body)
    return k(idx_flat, w_flat, emb)


_MLP_BLK = 1024


def _mlp_body(e_ref, w1_ref, b1_ref, w2_ref, b2_ref, o_ref):
    e = e_ref[...]
    h = lax.dot_general(e, w1_ref[...], (((1,), (1,)), ((), ())),
                        preferred_element_type=jnp.float32) + b1_ref[...]
    l = lax.dot_general(h, w2_ref[...], (((1,), (1,)), ((), ())),
                        preferred_element_type=jnp.float32) + b2_ref[...]
    m = jnp.max(l, axis=-1, keepdims=True)
    lse = jnp.log(jnp.sum(jnp.exp(l - m), axis=-1, keepdims=True)) + m
    o_ref[...] = l - lse


def _mlp(embed, W1, b1, W2, b2):
    grid = (_BATCH // _MLP_BLK,)
    return pl.pallas_call(
        _mlp_body,
        grid=grid,
        in_specs=[
            pl.BlockSpec((_MLP_BLK, _ORDER * _EMBED), lambda i: (i, 0)),
            pl.BlockSpec((_EMBED, _ORDER * _EMBED), lambda i: (0, 0)),
            pl.BlockSpec((1, _EMBED), lambda i: (0, 0)),
            pl.BlockSpec((_LABELS, _EMBED), lambda i: (0, 0)),
            pl.BlockSpec((1, _LABELS), lambda i: (0, 0)),
        ],
        out_specs=pl.BlockSpec((_MLP_BLK, _LABELS), lambda i: (i, 0)),
        out_shape=jax.ShapeDtypeStruct((_BATCH, _LABELS), jnp.float32),
    )(embed, W1, b1, W2, b2)


def kernel(ngrams, ngrams_weights, emb, W1, b1, W2, b2):
    idx_flat = ngrams.reshape(-1)
    w_flat = ngrams_weights.reshape(-1)
    embed = _sc_gather(idx_flat, w_flat, emb)
    return _mlp(embed, W1, b1.reshape(1, -1), W2, b2.reshape(1, -1))
